# Initial kernel scaffold; baseline (speedup 1.0000x reference)
import jax
import jax.numpy as jnp
from jax.experimental import pallas as pl


def kernel(tokens_buf, slot_ids_buf, logprobs_buf, num_tokens, finished, new_tokens, new_slot_ids, new_logprobs, num_new_tokens, finished_snapshot):
    raise NotImplementedError("write your pallas kernel here")


# speedup vs baseline: 1.6158x; 1.6158x over previous
import jax
import jax.numpy as jnp
from jax import lax
from jax.experimental import pallas as pl
from jax.experimental.pallas import tpu as pltpu
from jax.experimental.pallas import tpu_sc as plsc

MAX_TOKENS = 32768
MAX_SEQS = 128
NEW = 4096
NC = 2
NS = 16
NW = NC * NS
CHUNK = MAX_TOKENS // NW
VECS = CHUNK // 16

_mesh = plsc.VectorSubcoreMesh(core_axis_name="c", subcore_axis_name="s")


def _body(tok_hbm, sid_hbm, lp_hbm, t16_hbm, n16_hbm, fin_hbm, snap_hbm,
          ntok_hbm, nsid_hbm, nlp_hbm,
          out_tok, out_sid, out_lp, out_fin,
          t16_v, n16_v, oldt, olds, oldl, newt, news, newl, finv, snapv):
    wid = lax.axis_index("s") * NC + lax.axis_index("c")
    base = wid * CHUNK

    pltpu.sync_copy(t16_hbm, t16_v)
    pltpu.sync_copy(n16_hbm, n16_v)
    t_vec = t16_v[...]
    n_vec = n16_v[...]
    end_vec = t_vec + n_vec
    t_s = t_vec[0]
    n_s = n_vec[0]

    pltpu.sync_copy(tok_hbm.at[pl.ds(base, CHUNK)], oldt)
    pltpu.sync_copy(sid_hbm.at[pl.ds(base, CHUNK)], olds)
    pltpu.sync_copy(lp_hbm.at[pl.ds(base, CHUNK)], oldl)

    jlo = jnp.clip((t_s - base) >> 4, 0, VECS)
    jhi = jnp.clip((t_s + n_s - base + 15) >> 4, 0, VECS)

    @pl.when(jlo < jhi)
    def _patch():
        pltpu.sync_copy(ntok_hbm, newt)
        pltpu.sync_copy(nsid_hbm, news)
        pltpu.sync_copy(nlp_hbm, newl)

        iota = lax.iota(jnp.int32, 16)

        def jbody(j, carry):
            i0 = j * 16
            idx = base + i0 + iota
            m = (idx >= t_vec) & (idx < end_vec)
            off = jnp.clip(idx - t_vec, 0, NEW - 1)
            vt = plsc.load_gather(newt, [off])
            vs = plsc.load_gather(news, [off])
            vl = plsc.load_gather(newl, [off])
            oldt[pl.ds(i0, 16)] = jnp.where(m, vt, oldt[pl.ds(i0, 16)])
            olds[pl.ds(i0, 16)] = jnp.where(m, vs, olds[pl.ds(i0, 16)])
            oldl[pl.ds(i0, 16)] = jnp.where(m, vl, oldl[pl.ds(i0, 16)])
            return carry

        lax.fori_loop(jlo, jhi, jbody, 0)

    pltpu.sync_copy(oldt, out_tok.at[pl.ds(base, CHUNK)])
    pltpu.sync_copy(olds, out_sid.at[pl.ds(base, CHUNK)])
    pltpu.sync_copy(oldl, out_lp.at[pl.ds(base, CHUNK)])

    @pl.when(wid == 0)
    def _fin():
        pltpu.sync_copy(fin_hbm, finv)
        pltpu.sync_copy(snap_hbm, snapv)
        finv[pl.ds(0, 16)] = finv[pl.ds(0, 16)] | snapv[pl.ds(0, 16)]
        finv[pl.ds(16, 16)] = finv[pl.ds(16, 16)] | snapv[pl.ds(16, 16)]
        pltpu.sync_copy(finv, out_fin)


_sc_update = pl.kernel(
    _body,
    out_type=(
        jax.ShapeDtypeStruct((MAX_TOKENS,), jnp.int32),
        jax.ShapeDtypeStruct((MAX_TOKENS,), jnp.int32),
        jax.ShapeDtypeStruct((MAX_TOKENS,), jnp.float32),
        jax.ShapeDtypeStruct((MAX_SEQS // 4,), jnp.int32),
    ),
    mesh=_mesh,
    scratch_types=[
        pltpu.VMEM((16,), jnp.int32),
        pltpu.VMEM((16,), jnp.int32),
        pltpu.VMEM((CHUNK,), jnp.int32),
        pltpu.VMEM((CHUNK,), jnp.int32),
        pltpu.VMEM((CHUNK,), jnp.float32),
        pltpu.VMEM((NEW,), jnp.int32),
        pltpu.VMEM((NEW,), jnp.int32),
        pltpu.VMEM((NEW,), jnp.float32),
        pltpu.VMEM((MAX_SEQS // 4,), jnp.int32),
        pltpu.VMEM((MAX_SEQS // 4,), jnp.int32),
    ],
    compiler_params=pltpu.CompilerParams(needs_layout_passes=False),
)


def kernel(tokens_buf, slot_ids_buf, logprobs_buf, num_tokens, finished,
           new_tokens, new_slot_ids, new_logprobs, num_new_tokens,
           finished_snapshot):
    t = jnp.asarray(num_tokens, jnp.int32)
    n = jnp.asarray(num_new_tokens, jnp.int32)
    t16 = jnp.broadcast_to(t, (16,))
    n16 = jnp.broadcast_to(n, (16,))
    fin_w = lax.bitcast_convert_type(
        finished.astype(jnp.uint8).reshape(MAX_SEQS // 4, 4), jnp.int32)
    snap_w = lax.bitcast_convert_type(
        finished_snapshot.astype(jnp.uint8).reshape(MAX_SEQS // 4, 4), jnp.int32)
    out_tok, out_sid, out_lp, out_fin = _sc_update(
        tokens_buf, slot_ids_buf, logprobs_buf, t16, n16, fin_w, snap_w,
        new_tokens, new_slot_ids, new_logprobs)
    fin_bool = lax.bitcast_convert_type(out_fin, jnp.uint8).reshape(MAX_SEQS)
    return (out_tok, out_sid, out_lp, t + n, fin_bool.astype(jnp.bool_))


# speedup vs baseline: 1.8979x; 1.1746x over previous
import jax
import jax.numpy as jnp
from jax import lax
from jax.experimental import pallas as pl
from jax.experimental.pallas import tpu as pltpu
from jax.experimental.pallas import tpu_sc as plsc

MAX_TOKENS = 32768
MAX_SEQS = 128
NEW = 4096
NC = 2
NS = 16
NW = NC * NS
CHUNK = MAX_TOKENS // NW
VECS = CHUNK // 16
FW = MAX_SEQS // 4
SCAL = 32 + 2 * FW

_mesh = plsc.VectorSubcoreMesh(core_axis_name="c", subcore_axis_name="s")


def _body(tok_hbm, sid_hbm, lp_hbm, scal_hbm,
          ntok_hbm, nsid_hbm, nlp_hbm,
          out_tok, out_sid, out_lp, out_fin,
          scal_v, oldt, olds, oldl, newt, news, newl, finv,
          sem_scal, sem_in, sem_new, sem_out, sem_fin):
    wid = lax.axis_index("s") * NC + lax.axis_index("c")
    base = wid * CHUNK

    d_scal = pltpu.async_copy(scal_hbm, scal_v, sem_scal)
    d_t = pltpu.async_copy(tok_hbm.at[pl.ds(base, CHUNK)], oldt, sem_in)
    d_s = pltpu.async_copy(sid_hbm.at[pl.ds(base, CHUNK)], olds, sem_in)
    d_l = pltpu.async_copy(lp_hbm.at[pl.ds(base, CHUNK)], oldl, sem_in)

    d_scal.wait()
    t_vec = scal_v[pl.ds(0, 16)]
    n_vec = scal_v[pl.ds(16, 16)]
    end_vec = t_vec + n_vec
    t_s = t_vec[0]
    n_s = n_vec[0]

    jlo = jnp.clip((t_s - base) >> 4, 0, VECS)
    jhi = jnp.clip((t_s + n_s - base + 15) >> 4, 0, VECS)

    @pl.when(wid == 0)
    def _fin():
        finv[pl.ds(0, 16)] = scal_v[pl.ds(32, 16)] | scal_v[pl.ds(64, 16)]
        finv[pl.ds(16, 16)] = scal_v[pl.ds(48, 16)] | scal_v[pl.ds(80, 16)]
        pltpu.async_copy(finv, out_fin, sem_fin).wait()

    d_t.wait()
    d_s.wait()
    d_l.wait()

    @pl.when(jlo < jhi)
    def _patch():
        dn_t = pltpu.async_copy(ntok_hbm, newt, sem_new)
        dn_s = pltpu.async_copy(nsid_hbm, news, sem_new)
        dn_l = pltpu.async_copy(nlp_hbm, newl, sem_new)
        dn_t.wait()
        dn_s.wait()
        dn_l.wait()

        iota = lax.iota(jnp.int32, 16)

        def jbody(j, carry):
            i0 = j * 16
            idx = base + i0 + iota
            m = (idx >= t_vec) & (idx < end_vec)
            off = jnp.clip(idx - t_vec, 0, NEW - 1)
            vt = plsc.load_gather(newt, [off])
            vs = plsc.load_gather(news, [off])
            vl = plsc.load_gather(newl, [off])
            oldt[pl.ds(i0, 16)] = jnp.where(m, vt, oldt[pl.ds(i0, 16)])
            olds[pl.ds(i0, 16)] = jnp.where(m, vs, olds[pl.ds(i0, 16)])
            oldl[pl.ds(i0, 16)] = jnp.where(m, vl, oldl[pl.ds(i0, 16)])
            return carry

        lax.fori_loop(jlo, jhi, jbody, 0)

    do_t = pltpu.async_copy(oldt, out_tok.at[pl.ds(base, CHUNK)], sem_out)
    do_s = pltpu.async_copy(olds, out_sid.at[pl.ds(base, CHUNK)], sem_out)
    do_l = pltpu.async_copy(oldl, out_lp.at[pl.ds(base, CHUNK)], sem_out)
    do_t.wait()
    do_s.wait()
    do_l.wait()


_sc_update = pl.kernel(
    _body,
    out_type=(
        jax.ShapeDtypeStruct((MAX_TOKENS,), jnp.int32),
        jax.ShapeDtypeStruct((MAX_TOKENS,), jnp.int32),
        jax.ShapeDtypeStruct((MAX_TOKENS,), jnp.float32),
        jax.ShapeDtypeStruct((FW,), jnp.int32),
    ),
    mesh=_mesh,
    scratch_types=[
        pltpu.VMEM((SCAL,), jnp.int32),
        pltpu.VMEM((CHUNK,), jnp.int32),
        pltpu.VMEM((CHUNK,), jnp.int32),
        pltpu.VMEM((CHUNK,), jnp.float32),
        pltpu.VMEM((NEW,), jnp.int32),
        pltpu.VMEM((NEW,), jnp.int32),
        pltpu.VMEM((NEW,), jnp.float32),
        pltpu.VMEM((2 * 16,), jnp.int32),
        pltpu.SemaphoreType.DMA,
        pltpu.SemaphoreType.DMA,
        pltpu.SemaphoreType.DMA,
        pltpu.SemaphoreType.DMA,
        pltpu.SemaphoreType.DMA,
    ],
    compiler_params=pltpu.CompilerParams(needs_layout_passes=False),
)


def kernel(tokens_buf, slot_ids_buf, logprobs_buf, num_tokens, finished,
           new_tokens, new_slot_ids, new_logprobs, num_new_tokens,
           finished_snapshot):
    t = jnp.asarray(num_tokens, jnp.int32)
    n = jnp.asarray(num_new_tokens, jnp.int32)
    fin_w = lax.bitcast_convert_type(
        finished.astype(jnp.uint8).reshape(FW, 4), jnp.int32)
    snap_w = lax.bitcast_convert_type(
        finished_snapshot.astype(jnp.uint8).reshape(FW, 4), jnp.int32)
    scal = jnp.concatenate(
        [jnp.broadcast_to(t, (16,)), jnp.broadcast_to(n, (16,)),
         fin_w, snap_w])
    out_tok, out_sid, out_lp, out_fin = _sc_update(
        tokens_buf, slot_ids_buf, logprobs_buf, scal,
        new_tokens, new_slot_ids, new_logprobs)
    fin_bool = lax.bitcast_convert_type(out_fin, jnp.uint8).reshape(MAX_SEQS)
    return (out_tok, out_sid, out_lp, t + n, fin_bool.astype(jnp.bool_))


# speedup vs baseline: 1.9755x; 1.0409x over previous
import jax
import jax.numpy as jnp
from jax import lax
from jax.experimental import pallas as pl
from jax.experimental.pallas import tpu as pltpu
from jax.experimental.pallas import tpu_sc as plsc

MAX_TOKENS = 32768
MAX_SEQS = 128
NEW = 4096
NC = 1
NS = 16
NW = NC * NS
CHUNK = MAX_TOKENS // NW
VECS = CHUNK // 16
FW = MAX_SEQS // 4
SCAL = 32 + 2 * FW

_mesh = plsc.VectorSubcoreMesh(core_axis_name="c", subcore_axis_name="s",
                               num_cores=NC)


def _body(tok_hbm, sid_hbm, lp_hbm, scal_hbm,
          ntok_hbm, nsid_hbm, nlp_hbm,
          out_tok, out_sid, out_lp, out_fin,
          scal_v, oldt, olds, oldl, newt, news, newl, finv,
          sem_scal, sem_in, sem_new, sem_out, sem_fin):
    wid = lax.axis_index("s") * NC + lax.axis_index("c")
    base = wid * CHUNK

    d_scal = pltpu.async_copy(scal_hbm, scal_v, sem_scal)
    d_t = pltpu.async_copy(tok_hbm.at[pl.ds(base, CHUNK)], oldt, sem_in)
    d_s = pltpu.async_copy(sid_hbm.at[pl.ds(base, CHUNK)], olds, sem_in)
    d_l = pltpu.async_copy(lp_hbm.at[pl.ds(base, CHUNK)], oldl, sem_in)

    d_scal.wait()
    t_vec = scal_v[pl.ds(0, 16)]
    n_vec = scal_v[pl.ds(16, 16)]
    end_vec = t_vec + n_vec
    t_s = t_vec[0]
    n_s = n_vec[0]

    jlo = jnp.clip((t_s - base) >> 4, 0, VECS)
    jhi = jnp.clip((t_s + n_s - base + 15) >> 4, 0, VECS)

    @pl.when(wid == 0)
    def _fin():
        finv[pl.ds(0, 16)] = scal_v[pl.ds(32, 16)] | scal_v[pl.ds(64, 16)]
        finv[pl.ds(16, 16)] = scal_v[pl.ds(48, 16)] | scal_v[pl.ds(80, 16)]
        pltpu.async_copy(finv, out_fin, sem_fin).wait()

    d_t.wait()
    d_s.wait()
    d_l.wait()

    @pl.when(jlo < jhi)
    def _patch():
        dn_t = pltpu.async_copy(ntok_hbm, newt, sem_new)
        dn_s = pltpu.async_copy(nsid_hbm, news, sem_new)
        dn_l = pltpu.async_copy(nlp_hbm, newl, sem_new)
        dn_t.wait()
        dn_s.wait()
        dn_l.wait()

        iota = lax.iota(jnp.int32, 16)

        def jbody(j, carry):
            i0 = j * 16
            idx = base + i0 + iota
            m = (idx >= t_vec) & (idx < end_vec)
            off = jnp.clip(idx - t_vec, 0, NEW - 1)
            vt = plsc.load_gather(newt, [off])
            vs = plsc.load_gather(news, [off])
            vl = plsc.load_gather(newl, [off])
            oldt[pl.ds(i0, 16)] = jnp.where(m, vt, oldt[pl.ds(i0, 16)])
            olds[pl.ds(i0, 16)] = jnp.where(m, vs, olds[pl.ds(i0, 16)])
            oldl[pl.ds(i0, 16)] = jnp.where(m, vl, oldl[pl.ds(i0, 16)])
            return carry

        lax.fori_loop(jlo, jhi, jbody, 0)

    do_t = pltpu.async_copy(oldt, out_tok.at[pl.ds(base, CHUNK)], sem_out)
    do_s = pltpu.async_copy(olds, out_sid.at[pl.ds(base, CHUNK)], sem_out)
    do_l = pltpu.async_copy(oldl, out_lp.at[pl.ds(base, CHUNK)], sem_out)
    do_t.wait()
    do_s.wait()
    do_l.wait()


_sc_update = pl.kernel(
    _body,
    out_type=(
        jax.ShapeDtypeStruct((MAX_TOKENS,), jnp.int32),
        jax.ShapeDtypeStruct((MAX_TOKENS,), jnp.int32),
        jax.ShapeDtypeStruct((MAX_TOKENS,), jnp.float32),
        jax.ShapeDtypeStruct((FW,), jnp.int32),
    ),
    mesh=_mesh,
    scratch_types=[
        pltpu.VMEM((SCAL,), jnp.int32),
        pltpu.VMEM((CHUNK,), jnp.int32),
        pltpu.VMEM((CHUNK,), jnp.int32),
        pltpu.VMEM((CHUNK,), jnp.float32),
        pltpu.VMEM((NEW,), jnp.int32),
        pltpu.VMEM((NEW,), jnp.int32),
        pltpu.VMEM((NEW,), jnp.float32),
        pltpu.VMEM((2 * 16,), jnp.int32),
        pltpu.SemaphoreType.DMA,
        pltpu.SemaphoreType.DMA,
        pltpu.SemaphoreType.DMA,
        pltpu.SemaphoreType.DMA,
        pltpu.SemaphoreType.DMA,
    ],
    compiler_params=pltpu.CompilerParams(needs_layout_passes=False),
)


def kernel(tokens_buf, slot_ids_buf, logprobs_buf, num_tokens, finished,
           new_tokens, new_slot_ids, new_logprobs, num_new_tokens,
           finished_snapshot):
    t = jnp.asarray(num_tokens, jnp.int32)
    n = jnp.asarray(num_new_tokens, jnp.int32)
    fin_w = lax.bitcast_convert_type(
        finished.astype(jnp.uint8).reshape(FW, 4), jnp.int32)
    snap_w = lax.bitcast_convert_type(
        finished_snapshot.astype(jnp.uint8).reshape(FW, 4), jnp.int32)
    scal = jnp.concatenate(
        [jnp.broadcast_to(t, (16,)), jnp.broadcast_to(n, (16,)),
         fin_w, snap_w])
    out_tok, out_sid, out_lp, out_fin = _sc_update(
        tokens_buf, slot_ids_buf, logprobs_buf, scal,
        new_tokens, new_slot_ids, new_logprobs)
    fin_bool = lax.bitcast_convert_type(out_fin, jnp.uint8).reshape(MAX_SEQS)
    return (out_tok, out_sid, out_lp, t + n, fin_bool.astype(jnp.bool_))
